# baseline (speedup 1.0000x reference)
import functools

import jax
import jax.numpy as jnp
from jax.experimental import pallas as pl
from jax.experimental.pallas import tpu as pltpu

D = 768
N = 4 * 8192
RB = 256


def _tc_gather_add(x2d, idx2d, table):
    def body(idx_ref, x_ref, table_ref, o_ref):
        def row(r, _):
            t = idx_ref[0, r]
            o_ref[pl.ds(r, 1), :] = x_ref[pl.ds(r, 1), :] + table_ref[pl.ds(t, 1), :]
            return _
        jax.lax.fori_loop(0, RB, row, 0)

    return pl.pallas_call(
        body,
        grid=(N // RB,),
        in_specs=[
            pl.BlockSpec((1, RB), lambda i: (0, i),
                         memory_space=pltpu.SMEM),
            pl.BlockSpec((RB, D), lambda i: (i, 0)),
            pl.BlockSpec((8192, D), lambda i: (0, 0)),
        ],
        out_specs=pl.BlockSpec((RB, D), lambda i: (i, 0)),
        out_shape=jax.ShapeDtypeStruct((N, D), jnp.float32),
    )(idx2d, x2d, table)


def kernel(x, pe_index, pe_weight):
    b, s, d = x.shape
    x2d = x.reshape(N, D)
    idx2d = pe_index.reshape(1, N).astype(jnp.int32)
    out = _tc_gather_add(x2d, idx2d, pe_weight)
    return out.reshape(b, s, d)


# speedup vs baseline: 1.2267x; 1.2267x over previous
import functools

import jax
import jax.numpy as jnp
from jax import lax
from jax.experimental import pallas as pl
from jax.experimental.pallas import tpu as pltpu
from jax.experimental.pallas import tpu_sc as plsc

D = 768
N = 4 * 8192
V = 8192
N_TC = 8192
N_SC = N - N_TC
NC, NS = 2, 16
NW = NC * NS
PER_W = N_SC // NW
C = 16
NCH = PER_W // C
NB = 4
PD = 2
RB = 256


def _sc_fused(idx3d, x2d, table):
    mesh = plsc.VectorSubcoreMesh(core_axis_name="c", subcore_axis_name="s")

    @functools.partial(
        pl.kernel,
        out_type=jax.ShapeDtypeStruct((N_SC, D), jnp.float32),
        mesh=mesh,
        scratch_types=[
            pltpu.VMEM((NCH, C), jnp.int32),
            pltpu.VMEM((NB, C, D), jnp.float32),
            pltpu.VMEM((NB, C, D), jnp.float32),
            pltpu.SemaphoreType.DMA((NB,)),
            pltpu.SemaphoreType.DMA((NB,)),
            pltpu.SemaphoreType.DMA((NB,)),
        ],
    )
    def fused_kernel(idx_hbm, x_hbm, table_hbm, out_hbm,
                     idx_v, xb, rb, semx, semg, semo):
        wid = lax.axis_index("s") * NC + lax.axis_index("c")
        base = wid * PER_W
        pltpu.sync_copy(idx_hbm.at[wid], idx_v)

        def rows(t):
            return pl.ds(base + t * C, C)

        def start_in(t, b):
            pltpu.async_copy(x_hbm.at[rows(t)], xb.at[b], semx.at[b])
            pltpu.async_copy(table_hbm.at[idx_v.at[t]], rb.at[b], semg.at[b])

        def wait_in(t, b):
            pltpu.make_async_copy(x_hbm.at[rows(t)], xb.at[b], semx.at[b]).wait()
            pltpu.make_async_copy(
                table_hbm.at[idx_v.at[t]], rb.at[b], semg.at[b]).wait()

        def start_out(t, b):
            pltpu.async_copy(xb.at[b], out_hbm.at[rows(t)], semo.at[b])

        def wait_out(t, b):
            pltpu.make_async_copy(xb.at[b], out_hbm.at[rows(t)], semo.at[b]).wait()

        def add(b):
            @pl.loop(0, C)
            def _(r):
                @plsc.parallel_loop(0, D, step=16, unroll=8)
                def _(j):
                    sl = pl.ds(j, 16)
                    plsc.addupdate(xb.at[b, r, sl], rb[b, r, sl])

        for t in range(PD):
            start_in(t, t % NB)

        @pl.loop(0, NCH, step=NB)
        def _(t0):
            for k in range(NB):
                t = t0 + k
                b = k
                bp = (k + PD) % NB

                @pl.when(jnp.logical_and(t + PD < NCH, t + PD - NB >= 0))
                def _():
                    wait_out(t + PD - NB, bp)

                @pl.when(t + PD < NCH)
                def _():
                    start_in(t + PD, bp)

                wait_in(t, b)
                add(b)
                start_out(t, b)

        for t in range(NCH - NB, NCH):
            wait_out(t, t % NB)

    return fused_kernel(idx3d, x2d, table)


def _tc_gather_add(x2d, idx2d, table):
    def body(idx_ref, x_ref, table_ref, o_ref):
        def row(r, _):
            t = idx_ref[0, r]
            o_ref[pl.ds(r, 1), :] = x_ref[pl.ds(r, 1), :] + table_ref[pl.ds(t, 1), :]
            return _
        jax.lax.fori_loop(0, RB, row, 0)

    return pl.pallas_call(
        body,
        grid=(N_TC // RB,),
        in_specs=[
            pl.BlockSpec((1, RB), lambda i: (0, i),
                         memory_space=pltpu.SMEM),
            pl.BlockSpec((RB, D), lambda i: (i, 0)),
            pl.BlockSpec((V, D), lambda i: (0, 0)),
        ],
        out_specs=pl.BlockSpec((RB, D), lambda i: (i, 0)),
        out_shape=jax.ShapeDtypeStruct((N_TC, D), jnp.float32),
    )(idx2d, x2d, table)


def kernel(x, pe_index, pe_weight):
    b, s, d = x.shape
    x2d = x.reshape(N, D)
    idx = pe_index.reshape(N).astype(jnp.int32)
    idx3d = idx[:N_SC].reshape(NW, NCH, C)
    head = _sc_fused(idx3d, x2d[:N_SC], pe_weight)
    tail = _tc_gather_add(x2d[N_SC:], idx[N_SC:].reshape(1, N_TC), pe_weight)
    out = jnp.concatenate([head, tail], axis=0)
    return out.reshape(b, s, d)


# speedup vs baseline: 2.5411x; 2.0715x over previous
import functools

import jax
import jax.numpy as jnp
from jax import lax
from jax.experimental import pallas as pl
from jax.experimental.pallas import tpu as pltpu
from jax.experimental.pallas import tpu_sc as plsc

D = 768
N = 4 * 8192
NC, NS = 2, 16
NW = NC * NS
PER_W = N // NW
C = 32
NCH = PER_W // C


def _sc_fused(idx3d, x2d, table):
    mesh = plsc.VectorSubcoreMesh(core_axis_name="c", subcore_axis_name="s")

    @functools.partial(
        pl.kernel,
        out_type=jax.ShapeDtypeStruct((N, D), jnp.float32),
        mesh=mesh,
        scratch_types=[
            pltpu.VMEM((NCH, C), jnp.int32),
            pltpu.VMEM((2, C, D), jnp.float32),
            pltpu.VMEM((2, C, D), jnp.float32),
            pltpu.SemaphoreType.DMA((2,)),
            pltpu.SemaphoreType.DMA((2,)),
            pltpu.SemaphoreType.DMA((2,)),
        ],
    )
    def fused_kernel(idx_hbm, x_hbm, table_hbm, out_hbm,
                     idx_v, xb, rb, semx, semg, semo):
        wid = lax.axis_index("s") * NC + lax.axis_index("c")
        base = wid * PER_W
        pltpu.sync_copy(idx_hbm.at[wid], idx_v)

        def rows(t):
            return pl.ds(base + t * C, C)

        def start_in(t, p):
            pltpu.async_copy(x_hbm.at[rows(t)], xb.at[p], semx.at[p])
            pltpu.async_copy(table_hbm.at[idx_v.at[t]], rb.at[p], semg.at[p])

        def wait_in(t, p):
            pltpu.make_async_copy(x_hbm.at[rows(t)], xb.at[p], semx.at[p]).wait()
            pltpu.make_async_copy(
                table_hbm.at[idx_v.at[t]], rb.at[p], semg.at[p]).wait()

        def start_out(t, p):
            pltpu.async_copy(xb.at[p], out_hbm.at[rows(t)], semo.at[p])

        def wait_out(t, p):
            pltpu.make_async_copy(xb.at[p], out_hbm.at[rows(t)], semo.at[p]).wait()

        def add(p):
            @pl.loop(0, C)
            def _(r):
                @plsc.parallel_loop(0, D, step=16, unroll=4)
                def _(j):
                    sl = pl.ds(j, 16)
                    xb[p, r, sl] = xb[p, r, sl] + rb[p, r, sl]

        start_in(0, 0)

        @pl.loop(0, NCH, step=2)
        def _(t0):
            for p in (0, 1):
                t = t0 + p
                q = 1 - p

                @pl.when(t > 0)
                def _():
                    wait_out(t - 1, q)

                @pl.when(t + 1 < NCH)
                def _():
                    start_in(t + 1, q)

                wait_in(t, p)
                add(p)
                start_out(t, p)

        wait_out(NCH - 1, (NCH - 1) % 2)

    return fused_kernel(idx3d, x2d, table)


def kernel(x, pe_index, pe_weight):
    b, s, d = x.shape
    x2d = x.reshape(N, D)
    idx3d = pe_index.reshape(NW, NCH, C).astype(jnp.int32)
    out = _sc_fused(idx3d, x2d, pe_weight)
    return out.reshape(b, s, d)
